# baseline (speedup 1.0000x reference)
import functools

import jax
import jax.numpy as jnp
from jax import lax
from jax.experimental import pallas as pl
from jax.experimental.pallas import tpu as pltpu
from jax.experimental.pallas import tpu_sc as plsc

_NUM_WORKERS = 32
_BB = 512
_LANES = 16


@functools.partial(jax.jit, static_argnames=("v", "dim"))
def _detile_call(wt, wtail, *, v, dim):
    ncols = v // 128
    tail = v - ncols * 128
    mesh = plsc.VectorSubcoreMesh(core_axis_name="c", subcore_axis_name="s")

    @functools.partial(
        pl.kernel,
        mesh=mesh,
        out_type=jax.ShapeDtypeStruct((v * dim,), jnp.float32),
        scratch_types=[
            pltpu.VMEM((dim, 512), jnp.float32),
            pltpu.VMEM((dim, 512), jnp.float32),
            pltpu.VMEM((512 * dim,), jnp.float32),
            pltpu.VMEM((512 * dim,), jnp.float32),
            pltpu.SemaphoreType.DMA,
            pltpu.SemaphoreType.DMA,
        ],
        compiler_params=pltpu.CompilerParams(needs_layout_passes=False),
    )
    def detile(wt_hbm, wtail_hbm, tab_hbm, wbuf0, wbuf1, stg0, stg1, s_in, s_out):
        wid = lax.axis_index("s") * 2 + lax.axis_index("c")
        iot = lax.iota(jnp.int32, _LANES)
        iot_d = iot * dim
        wbufs = (wbuf0, wbuf1)
        stgs = (stg0, stg1)

        ngrp = ncols // 4
        cnt = jnp.where(wid < (ngrp % _NUM_WORKERS),
                        ngrp // _NUM_WORKERS + 1, ngrp // _NUM_WORKERS)

        def col_of(s):
            return wid + jnp.minimum(s, cnt - 1) * _NUM_WORKERS

        def start_in(j, buf):
            return pltpu.async_copy(
                wt_hbm.at[pl.ds(0, dim),
                          pl.ds(pl.multiple_of(j * 512, 512), 512)],
                wbufs[buf], s_in)

        def wait_in(buf):
            pltpu.make_async_copy(
                wt_hbm.at[pl.ds(0, dim), pl.ds(0, 512)],
                wbufs[buf], s_in).wait()

        def start_out(j, buf):
            return pltpu.async_copy(
                stgs[buf],
                tab_hbm.at[pl.ds(pl.multiple_of(j * 512 * dim, 512 * dim),
                                 512 * dim)], s_out)

        def wait_out(buf):
            pltpu.make_async_copy(
                stgs[buf],
                tab_hbm.at[pl.ds(0, 512 * dim)], s_out).wait()

        def transpose_col(buf):
            wb = wbufs[buf]
            sb = stgs[buf]

            @plsc.parallel_loop(0, dim * 32, unroll=8)
            def _(k):
                c = k // 32
                r0 = (k % 32) * _LANES
                vv = wb[c, pl.ds(r0, _LANES)]
                plsc.store_scatter(sb, [iot_d + (r0 * dim + c)], vv)

        start_in(col_of(0), 0)
        nslots = (ngrp // _NUM_WORKERS + 2 + 1) // 2 * 2
        def pair_body(p, carry):
            for sub in (0, 1):
                buf = sub
                s_cur = 2 * p + sub
                wait_in(buf)
                start_in(col_of(s_cur + 1), buf ^ 1)

                @pl.when(s_cur >= 2)
                def _():
                    wait_out(buf)
                transpose_col(buf)
                start_out(col_of(s_cur), buf)
            return carry

        lax.fori_loop(0, nslots // 2, pair_body, 0)
        wait_in(0)
        wait_out(0)
        wait_out(1)

        if tail:
            @pl.when(wid == _NUM_WORKERS - 1)
            def _():
                pltpu.sync_copy(wtail_hbm, stg0.at[pl.ds(0, tail * dim)])
                pltpu.sync_copy(
                    stg0.at[pl.ds(0, tail * dim)],
                    tab_hbm.at[pl.ds((v - tail) * dim, tail * dim)])

    return detile(wt, wtail)


@functools.partial(jax.jit, static_argnames=("b", "f", "dim"))
def _gather_call(idxf, table, *, b, f, dim):
    n_total = b * f
    nbb = b // _BB
    n_chunks = f * nbb
    per_w = n_chunks // _NUM_WORKERS
    ntc = dim // 8
    ntbl = _BB // 128
    npairs = _BB * dim // _LANES
    mesh = plsc.VectorSubcoreMesh(core_axis_name="c", subcore_axis_name="s")

    @functools.partial(
        pl.kernel,
        mesh=mesh,
        out_type=jax.ShapeDtypeStruct((n_total * dim,), jnp.float32),
        scratch_types=[
            pltpu.VMEM((2, _BB), jnp.int32),
            pltpu.VMEM((2, _BB, dim), jnp.float32),
            pltpu.VMEM((2, _BB * dim), jnp.float32),
            pltpu.SemaphoreType.DMA,
            pltpu.SemaphoreType.DMA,
            pltpu.SemaphoreType.DMA,
        ],
        compiler_params=pltpu.CompilerParams(
            use_tc_tiling_on_sc=False, needs_layout_passes=False),
    )
    def emb(idx_hbm, table_hbm, out_hbm, ibuf, rows, stg, si, sg, so):
        wid = lax.axis_index("s") * 2 + lax.axis_index("c")
        iot = lax.iota(jnp.int32, _LANES)

        def chunk_id(s):
            return wid + s * _NUM_WORKERS

        def idx_off(s):
            n = chunk_id(s)
            fi, bb = n // nbb, n % nbb
            return pl.multiple_of(fi * b + bb * _BB, _BB)

        def start_idx(s, buf):
            return pltpu.async_copy(
                idx_hbm.at[pl.ds(idx_off(s), _BB)], ibuf.at[buf], si)

        def start_gather(s, buf):
            return pltpu.async_copy(
                table_hbm.at[ibuf.at[buf]], rows.at[buf], sg)

        base0 = (iot // 8) * (_BB * 8) + (iot % 8) * 128
        bases = tuple(base0 + jl for jl in range(8))
        span = (_LANES // 8 - 1) * (_BB * 8) + 7 * 128 + 8
        half = _BB * 8 * 2

        def transpose_chunk(buf):
            rbuf = rows.at[buf]
            sbuf = stg.at[buf]

            @plsc.parallel_loop(0, _BB // 8, unroll=2)
            def _(q):
                jbase = q * 8
                soff = (jbase // 128) * 1024 + (jbase % 128)
                for jl in range(8):
                    v0 = rbuf[jbase + jl, pl.ds(0, _LANES)]
                    v1 = rbuf[jbase + jl, pl.ds(_LANES, _LANES)]
                    plsc.store_scatter(
                        sbuf.at[pl.ds(soff, span)], [bases[jl]], v0)
                    plsc.store_scatter(
                        sbuf.at[pl.ds(soff + half, span)], [bases[jl]], v1)

        def start_out(s, buf):
            n = chunk_id(s)
            fi, bb = n // nbb, n % nbb
            handles = []
            for tc in range(ntc):
                off = pl.multiple_of(
                    (fi * ntc + tc) * (b * 8) + bb * (_BB * 8), _BB * 8)
                handles.append(pltpu.async_copy(
                    stg.at[buf].at[pl.ds(tc * (_BB * 8), _BB * 8)],
                    out_hbm.at[pl.ds(off, _BB * 8)], so))
            return handles

        idx_cp = [None] * per_w
        g_cp = [None] * per_w
        o_cp = [None] * per_w
        pltpu.sync_copy(idx_hbm.at[pl.ds(idx_off(0), _BB)], ibuf.at[0])
        g_cp[0] = start_gather(0, 0)
        if per_w > 1:
            idx_cp[1] = start_idx(1, 1)
        for s in range(per_w):
            cur = s % 2
            if s + 1 < per_w:
                idx_cp[s + 1].wait()
                g_cp[s + 1] = start_gather(s + 1, cur ^ 1)
            g_cp[s].wait()
            if s + 2 < per_w:
                idx_cp[s + 2] = start_idx(s + 2, cur)
            if s >= 2:
                for h in o_cp[s - 2]:
                    h.wait()
            transpose_chunk(cur)
            o_cp[s] = start_out(s, cur)
        for s in (per_w - 2, per_w - 1):
            if s >= 0:
                for h in o_cp[s]:
                    h.wait()

    return emb(idxf, table)


def kernel(input, weight):
    b, f = input.shape
    v, dim = weight.shape
    tail = v % 128
    wtail = lax.slice(weight, (v - tail, 0), (v, dim)).reshape(tail * dim)
    table1d = _detile_call(weight.T, wtail, v=v, dim=dim)
    idxf = input.T.reshape(b * f).astype(jnp.int32)
    out1d = _gather_call(idxf, table1d.reshape(v, dim), b=b, f=f, dim=dim)
    t = out1d.reshape(f, dim // 8, b // 128, 8, 128)
    t = t.transpose(2, 4, 0, 1, 3)
    return t.reshape(b, f, dim)


# speedup vs baseline: 1.1939x; 1.1939x over previous
import functools

import jax
import jax.numpy as jnp
from jax import lax
from jax.experimental import pallas as pl
from jax.experimental.pallas import tpu as pltpu
from jax.experimental.pallas import tpu_sc as plsc

_NUM_WORKERS = 32
_BB = 512
_LANES = 16


@functools.partial(jax.jit, static_argnames=("b", "f", "dim"))
def _gather_call(idxf, table, *, b, f, dim):
    n_total = b * f
    nbb = b // _BB
    n_chunks = f * nbb
    per_w = n_chunks // _NUM_WORKERS
    ntc = dim // 8
    mesh = plsc.VectorSubcoreMesh(core_axis_name="c", subcore_axis_name="s")

    @functools.partial(
        pl.kernel,
        mesh=mesh,
        out_type=jax.ShapeDtypeStruct((n_total * dim,), jnp.float32),
        scratch_types=[
            pltpu.VMEM((2, _BB), jnp.int32),
            pltpu.VMEM((2, _BB, dim), jnp.float32),
            pltpu.VMEM((2, _BB * dim), jnp.float32),
            pltpu.SemaphoreType.DMA,
            pltpu.SemaphoreType.DMA,
            pltpu.SemaphoreType.DMA,
        ],
        compiler_params=pltpu.CompilerParams(
            use_tc_tiling_on_sc=False, needs_layout_passes=False),
    )
    def emb(idx_hbm, table_hbm, out_hbm, ibuf, rows, stg, si, sg, so):
        wid = lax.axis_index("s") * 2 + lax.axis_index("c")
        iot = lax.iota(jnp.int32, _LANES)

        def chunk_id(s):
            return wid + s * _NUM_WORKERS

        def idx_off(s):
            n = chunk_id(s)
            fi, bb = n // nbb, n % nbb
            return pl.multiple_of(fi * b + bb * _BB, _BB)

        def start_idx(s, buf):
            return pltpu.async_copy(
                idx_hbm.at[pl.ds(idx_off(s), _BB)], ibuf.at[buf], si)

        def start_gather(s, buf):
            return pltpu.async_copy(
                table_hbm.at[ibuf.at[buf]], rows.at[buf], sg)

        base0 = (iot // 8) * (_BB * 8) + (iot % 8) * 128
        base1 = ((iot + _LANES) // 8) * (_BB * 8) + ((iot + _LANES) % 8) * 128

        def transpose_chunk(buf):
            rbuf = rows.at[buf]
            sbuf = stg.at[buf]

            @plsc.parallel_loop(0, _BB, unroll=4)
            def _(j):
                off = (j // 128) * 1024 + (j % 128)
                v0 = rbuf[j, pl.ds(0, _LANES)]
                v1 = rbuf[j, pl.ds(_LANES, _LANES)]
                plsc.store_scatter(sbuf, [base0 + off], v0)
                plsc.store_scatter(sbuf, [base1 + off], v1)

        def start_out(s, buf):
            n = chunk_id(s)
            fi, bb = n // nbb, n % nbb
            handles = []
            for tc in range(ntc):
                off = pl.multiple_of(
                    (fi * ntc + tc) * (b * 8) + bb * (_BB * 8), _BB * 8)
                handles.append(pltpu.async_copy(
                    stg.at[buf].at[pl.ds(tc * (_BB * 8), _BB * 8)],
                    out_hbm.at[pl.ds(off, _BB * 8)], so))
            return handles

        idx_cp = [None] * per_w
        g_cp = [None] * per_w
        o_cp = [None] * per_w
        pltpu.sync_copy(idx_hbm.at[pl.ds(idx_off(0), _BB)], ibuf.at[0])
        g_cp[0] = start_gather(0, 0)
        if per_w > 1:
            idx_cp[1] = start_idx(1, 1)
        for s in range(per_w):
            cur = s % 2
            if s + 1 < per_w:
                idx_cp[s + 1].wait()
                g_cp[s + 1] = start_gather(s + 1, cur ^ 1)
            g_cp[s].wait()
            if s + 2 < per_w:
                idx_cp[s + 2] = start_idx(s + 2, cur)
            if s >= 2:
                for h in o_cp[s - 2]:
                    h.wait()
            transpose_chunk(cur)
            o_cp[s] = start_out(s, cur)
        for s in (per_w - 2, per_w - 1):
            if s >= 0:
                for h in o_cp[s]:
                    h.wait()

    return emb(idxf, table)


def kernel(input, weight):
    b, f = input.shape
    v, dim = weight.shape
    idxf = input.T.reshape(b * f).astype(jnp.int32)
    out1d = _gather_call(idxf, weight, b=b, f=f, dim=dim)
    t = out1d.reshape(f, dim // 8, b // 128, 8, 128)
    t = t.transpose(2, 4, 0, 1, 3)
    return t.reshape(b, f, dim)
